# Initial kernel scaffold; baseline (speedup 1.0000x reference)
import jax
import jax.numpy as jnp
from jax.experimental import pallas as pl


def kernel(x, W_down, W_up, W_gate, b_gate, W_ca3):
    raise NotImplementedError("write your pallas kernel here")


# speedup vs baseline: 11.9609x; 11.9609x over previous
import jax
import jax.numpy as jnp
from jax.experimental import pallas as pl
from jax.experimental.pallas import tpu as pltpu

SEQ = 2048
D_MODEL = 768
N_CA3 = 4096
K_WTA = 64
TN = 256
GRID = SEQ // TN
_NBITS = 16


def _kwta_normalize(h):
    ncols = h.shape[1]
    thr = jnp.zeros((1, ncols), dtype=jnp.int32)
    for b in range(30, 30 - _NBITS, -1):
        cand = thr | (1 << b)
        cand_f = jax.lax.bitcast_convert_type(cand, jnp.float32)
        cnt = jnp.sum((h >= cand_f).astype(jnp.int32), axis=0, keepdims=True)
        thr = jnp.where(cnt >= K_WTA, cand, thr)
    thr_f = jax.lax.bitcast_convert_type(thr, jnp.float32)
    s = jnp.where(h >= thr_f, h, 0.0)
    norm = jnp.sqrt(jnp.sum(s * s, axis=0, keepdims=True))
    return s * (1.0 / jnp.maximum(norm, 1e-10))


def _sparsify_body(wd_ref, xT_ref, sT_ref):
    h = jnp.dot(wd_ref[...], xT_ref[...].astype(jnp.bfloat16),
                preferred_element_type=jnp.float32)
    h = jnp.maximum(h, 0.0)
    sT_ref[...] = _kwta_normalize(h).astype(jnp.bfloat16)


def _retrieve_body(wc_ref, sT_ref, h2T_ref, tot_ref, acc_ref):
    i = pl.program_id(0)
    k = pl.program_id(1)
    nk = pl.num_programs(1)

    part = jnp.dot(wc_ref[...], sT_ref[...],
                   preferred_element_type=jnp.float32)

    @pl.when(k == 0)
    def _init_acc():
        acc_ref[...] = part

    @pl.when(k != 0)
    def _accum():
        acc_ref[...] += part

    @pl.when((i == 0) & (k == 0))
    def _init_tot():
        tot_ref[...] = jnp.zeros((1, 1), jnp.float32)

    @pl.when(k == nk - 1)
    def _finish():
        h2 = jnp.maximum(acc_ref[...], 0.0)
        tot_ref[...] += jnp.sum(h2).reshape(1, 1)
        h2T_ref[...] = h2.astype(jnp.bfloat16)


def _combine_body(xT_ref, sT_ref, h2T_ref, tot_ref, wu_ref, wg_ref, bg_ref,
                  outT_ref):
    xn = _kwta_normalize(h2T_ref[...].astype(jnp.float32))
    cond = tot_ref[...] < 1e-10
    succ = jnp.where(cond, sT_ref[...].astype(jnp.float32), xn)
    predT = jnp.dot(wu_ref[...], succ.astype(jnp.bfloat16),
                    preferred_element_type=jnp.float32)
    xT = xT_ref[...]
    zT = jnp.dot(wg_ref[...], xT.astype(jnp.bfloat16),
                 preferred_element_type=jnp.float32) + bg_ref[...]
    gT = jax.nn.sigmoid(zT)
    outT_ref[...] = xT + gT * predT


def kernel(x, W_down, W_up, W_gate, b_gate, W_ca3):
    xT = x.reshape(SEQ, D_MODEL).T
    wd = W_down.astype(jnp.bfloat16)
    wc = W_ca3.astype(jnp.bfloat16)
    wu = W_up.astype(jnp.bfloat16)
    wg = W_gate.astype(jnp.bfloat16)
    bg = b_gate.reshape(D_MODEL, 1)

    cp = pltpu.CompilerParams(vmem_limit_bytes=63 * 1024 * 1024)

    sT = pl.pallas_call(
        _sparsify_body,
        grid=(GRID,),
        in_specs=[
            pl.BlockSpec((N_CA3, D_MODEL), lambda i: (0, 0)),
            pl.BlockSpec((D_MODEL, TN), lambda i: (0, i)),
        ],
        out_specs=pl.BlockSpec((N_CA3, TN), lambda i: (0, i)),
        out_shape=jax.ShapeDtypeStruct((N_CA3, SEQ), jnp.bfloat16),
        compiler_params=cp,
    )(wd, xT)

    KP = 1024
    h2T, tot = pl.pallas_call(
        _retrieve_body,
        grid=(GRID, N_CA3 // KP),
        in_specs=[
            pl.BlockSpec((N_CA3, KP), lambda i, k: (0, k)),
            pl.BlockSpec((KP, TN), lambda i, k: (k, i)),
        ],
        out_specs=[
            pl.BlockSpec((N_CA3, TN), lambda i, k: (0, i)),
            pl.BlockSpec((1, 1), lambda i, k: (0, 0)),
        ],
        out_shape=[
            jax.ShapeDtypeStruct((N_CA3, SEQ), jnp.bfloat16),
            jax.ShapeDtypeStruct((1, 1), jnp.float32),
        ],
        scratch_shapes=[
            pltpu.VMEM((N_CA3, TN), jnp.float32),
        ],
        compiler_params=cp,
    )(wc, sT)

    outT = pl.pallas_call(
        _combine_body,
        grid=(GRID,),
        in_specs=[
            pl.BlockSpec((D_MODEL, TN), lambda i: (0, i)),
            pl.BlockSpec((N_CA3, TN), lambda i: (0, i)),
            pl.BlockSpec((N_CA3, TN), lambda i: (0, i)),
            pl.BlockSpec((1, 1), lambda i: (0, 0)),
            pl.BlockSpec((D_MODEL, N_CA3), lambda i: (0, 0)),
            pl.BlockSpec((D_MODEL, D_MODEL), lambda i: (0, 0)),
            pl.BlockSpec((D_MODEL, 1), lambda i: (0, 0)),
        ],
        out_specs=pl.BlockSpec((D_MODEL, TN), lambda i: (0, i)),
        out_shape=jax.ShapeDtypeStruct((D_MODEL, SEQ), jnp.float32),
        compiler_params=cp,
    )(xT, sT, h2T, tot, wu, wg, bg)

    return outT.T.reshape(1, SEQ, D_MODEL)
